# baseline (speedup 1.0000x reference)
import functools

import jax
import jax.numpy as jnp
from jax import lax
from jax.experimental import pallas as pl
from jax.experimental.pallas import tpu as pltpu
from jax.experimental.pallas import tpu_sc as plsc

EMBED = 1024
HEADS = 4
HP = 16
N_TOK = 32768
N_BAGS = 16
HIDDEN_PAD = 384
BLK = 1024
N_BLKS = N_TOK // BLK
BLK3 = 2048
N_BLKS3 = N_TOK // BLK3
CLIP = 1e-5

NW = 32
CHUNK = N_TOK // NW
ELEMS = CHUNK * HEADS
DMA_B = 128


def _logits_body(x_ref, seg_ref, wvu_ref, bvu_ref, wa_ref, ba_row_ref,
                 l16_ref, xb_ref, den_ref, offcnt_ref, den_s, oc_s):
    i = pl.program_id(0)
    x = x_ref[...].astype(jnp.bfloat16)
    xb_ref[...] = x
    pre = lax.dot_general(x, wvu_ref[...], (((1,), (1,)), ((), ())),
                          preferred_element_type=jnp.float32)
    pre = pre + bvu_ref[...]
    v = jnp.tanh(pre[:, :HIDDEN_PAD])
    u = jax.nn.sigmoid(pre[:, HIDDEN_PAD:])
    g = v * u
    l16 = lax.dot_general(g, wa_ref[...], (((1,), (0,)), ((), ())),
                          preferred_element_type=jnp.float32) + ba_row_ref[...]
    l16_ref[...] = l16

    @pl.when(i == 0)
    def _init_stats():
        den_s[...] = jnp.zeros_like(den_s)
        oc_s[...] = jnp.zeros_like(oc_s)

    seg = seg_ref[...]
    bag = lax.broadcasted_iota(jnp.int32, (BLK, N_BAGS), 1)
    onehot = (seg == bag).astype(jnp.float32)
    e = jnp.exp(l16)
    den_s[...] += lax.dot_general(e, onehot, (((0,), (0,)), ((), ())),
                                  preferred_element_type=jnp.float32)
    oc_s[...] += jnp.sum(onehot, axis=0, keepdims=True)

    @pl.when(i == N_BLKS - 1)
    def _emit_stats():
        d = den_s[...]
        den_ref[...] = jnp.where(d == 0.0, 1.0, d)
        cnt_bc = jnp.broadcast_to(oc_s[...], (N_BAGS, N_BAGS))
        r = lax.broadcasted_iota(jnp.int32, (N_BAGS, N_BAGS), 0)
        c = lax.broadcasted_iota(jnp.int32, (N_BAGS, N_BAGS), 1)
        off_col = jnp.sum(jnp.where(c < r, cnt_bc, 0.0), axis=1, keepdims=True)
        cnt_col = jnp.sum(jnp.where(c == r, cnt_bc, 0.0), axis=1, keepdims=True)
        offcnt_ref[...] = jnp.concatenate(
            [off_col, cnt_col], axis=1).astype(jnp.int32)


def _compute_logits(x, seg2d, wvu, bvu, wa16, ba_row):
    return pl.pallas_call(
        _logits_body,
        grid=(N_BLKS,),
        in_specs=[
            pl.BlockSpec((BLK, EMBED), lambda i: (i, 0)),
            pl.BlockSpec((BLK, 1), lambda i: (i, 0)),
            pl.BlockSpec((2 * HIDDEN_PAD, EMBED), lambda i: (0, 0)),
            pl.BlockSpec((1, 2 * HIDDEN_PAD), lambda i: (0, 0)),
            pl.BlockSpec((HIDDEN_PAD, HP), lambda i: (0, 0)),
            pl.BlockSpec((1, HP), lambda i: (0, 0)),
        ],
        out_specs=[
            pl.BlockSpec((BLK, HP), lambda i: (i, 0)),
            pl.BlockSpec((BLK, EMBED), lambda i: (i, 0)),
            pl.BlockSpec((HP, N_BAGS), lambda i: (0, 0)),
            pl.BlockSpec((N_BAGS, 2), lambda i: (0, 0)),
        ],
        out_shape=[
            jax.ShapeDtypeStruct((N_TOK, HP), jnp.float32),
            jax.ShapeDtypeStruct((N_TOK, EMBED), jnp.bfloat16),
            jax.ShapeDtypeStruct((HP, N_BAGS), jnp.float32),
            jax.ShapeDtypeStruct((N_BAGS, 2), jnp.int32),
        ],
        scratch_shapes=[
            pltpu.VMEM((HP, N_BAGS), jnp.float32),
            pltpu.VMEM((1, N_BAGS), jnp.float32),
        ],
    )(x, seg2d, wvu, bvu, wa16, ba_row)


def _pool_body(l16_ref, seg_ref, den_ref, x_ref, wm_ref, bm_ref, out_ref, acc_ref):
    i = pl.program_id(0)

    @pl.when(i == 0)
    def _init():
        acc_ref[...] = jnp.zeros_like(acc_ref)

    @pl.when(i < N_BLKS3)
    def _accum():
        l16 = l16_ref[...]
        seg = seg_ref[...]
        bag = lax.broadcasted_iota(jnp.int32, (BLK3, N_BAGS), 1)
        onehot = (seg == bag).astype(jnp.float32)
        tok_den = lax.dot_general(onehot, den_ref[...], (((1,), (1,)), ((), ())),
                                  preferred_element_type=jnp.float32)
        att16 = jnp.maximum(jnp.exp(l16) / tok_den, CLIP)
        p = lax.broadcasted_iota(jnp.int32, (HP, HEADS * N_BAGS), 0)
        q = lax.broadcasted_iota(jnp.int32, (HP, HEADS * N_BAGS), 1)
        expand = (p == q // N_BAGS).astype(jnp.float32)
        att64 = jnp.dot(att16, expand, preferred_element_type=jnp.float32)
        mask = jnp.concatenate([onehot] * HEADS, axis=1)
        a_mat = (att64 * mask).astype(jnp.bfloat16)
        acc_ref[...] += lax.dot_general(a_mat, x_ref[...], (((0,), (0,)), ((), ())),
                                        preferred_element_type=jnp.float32)

    @pl.when(i == N_BLKS3)
    def _final():
        acc = acc_ref[...]
        res = bm_ref[...]
        for h in range(HEADS):
            res += lax.dot_general(
                acc[h * N_BAGS:(h + 1) * N_BAGS, :],
                wm_ref[:, pl.ds(h * EMBED, EMBED)],
                (((1,), (1,)), ((), ())),
                preferred_element_type=jnp.float32)
        out_ref[...] = res


def _pool_project(l16, seg2d, den, xb, wm, bm2d):
    last = N_BLKS3 - 1
    return pl.pallas_call(
        _pool_body,
        grid=(N_BLKS3 + 1,),
        in_specs=[
            pl.BlockSpec((BLK3, HP), lambda i: (jnp.minimum(i, last), 0)),
            pl.BlockSpec((BLK3, 1), lambda i: (jnp.minimum(i, last), 0)),
            pl.BlockSpec((HP, N_BAGS), lambda i: (0, 0)),
            pl.BlockSpec((BLK3, EMBED), lambda i: (jnp.minimum(i, last), 0)),
            pl.BlockSpec((EMBED, HEADS * EMBED), lambda i: (0, 0)),
            pl.BlockSpec((N_BAGS, EMBED), lambda i: (0, 0)),
        ],
        out_specs=pl.BlockSpec((N_BAGS, EMBED), lambda i: (0, 0)),
        out_shape=jax.ShapeDtypeStruct((N_BAGS, EMBED), jnp.float32),
        scratch_shapes=[pltpu.VMEM((HEADS * N_BAGS, EMBED), jnp.float32)],
    )(l16, seg2d, den, xb, wm, bm2d)


def _w_body(l16_hbm, seg_hbm, off_hbm, cnt_hbm, den_hbm, w_hbm,
            seg_v, off_v, cnt_v, den_v, idx_v, col_v, d_v, rows_v, w_v, sem):
    c = lax.axis_index("c")
    s = lax.axis_index("s")
    wid = s * 2 + c
    base = wid * CHUNK
    pltpu.sync_copy(seg_hbm.at[pl.ds(base, CHUNK)], seg_v)
    pltpu.sync_copy(off_hbm, off_v)
    pltpu.sync_copy(cnt_hbm, cnt_v)
    pltpu.sync_copy(den_hbm, den_v)

    lane = lax.iota(jnp.int32, 16)

    def phase1(g, carry):
        i16 = g * 16 + lane
        q = i16 >> 2
        hh = i16 & 3
        sg = plsc.load_gather(seg_v, [q])
        off = plsc.load_gather(off_v, [sg])
        n = plsc.load_gather(cnt_v, [sg])
        k = (base + q - off) * HEADS + hh
        cdiv = k // n
        idx_v[pl.ds(g * 16, 16)] = off + (k - cdiv * n)
        col_v[pl.ds(g * 16, 16)] = cdiv
        d_v[pl.ds(g * 16, 16)] = plsc.load_gather(den_v, [cdiv, sg])
        return carry

    lax.fori_loop(0, ELEMS // 16, phase1, 0, unroll=4)

    def phase2(j, carry):
        pltpu.async_copy(l16_hbm.at[idx_v.at[pl.ds(j * DMA_B, DMA_B)]],
                         rows_v.at[pl.ds(j * DMA_B, DMA_B)], sem)
        return carry

    lax.fori_loop(0, ELEMS // DMA_B, phase2, 0, unroll=False)
    pltpu.make_async_copy(l16_hbm.at[pl.ds(0, ELEMS)], rows_v, sem).wait()

    def phase3(g, carry):
        i16 = g * 16 + lane
        cdiv = col_v[pl.ds(g * 16, 16)]
        lg = plsc.load_gather(rows_v, [i16, cdiv])
        d = d_v[pl.ds(g * 16, 16)]
        wv = jnp.maximum(jnp.exp(lg) / d, CLIP)
        plsc.store_scatter(w_v, [i16 & 3, i16 >> 2], wv)
        return carry

    lax.fori_loop(0, ELEMS // 16, phase3, 0, unroll=4)
    pltpu.sync_copy(w_v, w_hbm.at[:, pl.ds(base, CHUNK)])


@functools.lru_cache(maxsize=1)
def _get_w_kernel():
    @functools.partial(
        pl.kernel,
        mesh=plsc.VectorSubcoreMesh(core_axis_name="c", subcore_axis_name="s"),
        out_type=jax.ShapeDtypeStruct((HEADS, N_TOK), jnp.float32),
        compiler_params=pltpu.CompilerParams(
            needs_layout_passes=False, use_tc_tiling_on_sc=False),
        scratch_types=[
            pltpu.VMEM((CHUNK,), jnp.int32),
            pltpu.VMEM((N_BAGS,), jnp.int32),
            pltpu.VMEM((N_BAGS,), jnp.int32),
            pltpu.VMEM((HP, N_BAGS), jnp.float32),
            pltpu.VMEM((ELEMS,), jnp.int32),
            pltpu.VMEM((ELEMS,), jnp.int32),
            pltpu.VMEM((ELEMS,), jnp.float32),
            pltpu.VMEM((ELEMS, HP), jnp.float32),
            pltpu.VMEM((HEADS, CHUNK), jnp.float32),
            pltpu.SemaphoreType.DMA,
        ],
    )
    def _w_sc(l16, seg, off, cnt, den, w_out, *scratch):
        _w_body(l16, seg, off, cnt, den, w_out, *scratch)

    return _w_sc


def _w_sparsecore(l16, seg, off, cnt, den):
    return _get_w_kernel()(l16, seg, off, cnt, den)


def kernel(x, supercase_indices, Wv, bv, Wu, bu, Wa, ba, Wm, bm):
    seg = supercase_indices.astype(jnp.int32)
    seg2d = seg.reshape(N_TOK, 1)

    h = Wv.shape[0]
    pad = HIDDEN_PAD - h
    zrow = jnp.zeros((pad, EMBED), jnp.float32)
    wvu = jnp.concatenate([Wv, zrow, Wu, zrow],
                          axis=0).astype(jnp.bfloat16)
    zb = jnp.zeros((pad,), jnp.float32)
    bvu = jnp.concatenate([bv, zb, bu, zb]).reshape(1, 2 * HIDDEN_PAD)
    wa16 = jnp.zeros((HIDDEN_PAD, HP), jnp.float32).at[:h, :HEADS].set(Wa.T)
    ba_row = jnp.zeros((1, HP), jnp.float32).at[0, :HEADS].set(ba)
    bm2d = jnp.broadcast_to(bm.reshape(1, EMBED), (N_BAGS, EMBED))

    l16, xb, den, offcnt = _compute_logits(x, seg2d, wvu, bvu, wa16, ba_row)
    out = _pool_project(l16, seg2d, den, xb, Wm, bm2d)
    wt = _w_sparsecore(l16, seg, offcnt[:, 0], offcnt[:, 1], den)
    return (out, wt.T)


# speedup vs baseline: 1.1228x; 1.1228x over previous
import functools

import jax
import jax.numpy as jnp
from jax import lax
from jax.experimental import pallas as pl
from jax.experimental.pallas import tpu as pltpu
from jax.experimental.pallas import tpu_sc as plsc

EMBED = 1024
HEADS = 4
HP = 16
N_TOK = 32768
N_BAGS = 16
HIDDEN_PAD = 384
BLK = 1024
N_BLKS = N_TOK // BLK
BLK3 = 4096
N_BLKS3 = N_TOK // BLK3
CLIP = 1e-5

NW = 32
CHUNK = N_TOK // NW
ELEMS = CHUNK * HEADS
DMA_B = 128


def _logits_body(x_ref, seg_ref, wvu_ref, bvu_ref, wa_ref, ba_row_ref,
                 l16_ref, xb_ref, den_ref, offcnt_ref, den_s, oc_s):
    i = pl.program_id(0)
    x = x_ref[...].astype(jnp.bfloat16)
    xb_ref[...] = x
    pre = lax.dot_general(x, wvu_ref[...], (((1,), (1,)), ((), ())),
                          preferred_element_type=jnp.float32)
    pre = pre + bvu_ref[...]
    v = jnp.tanh(pre[:, :HIDDEN_PAD])
    u = jax.nn.sigmoid(pre[:, HIDDEN_PAD:])
    g = v * u
    l16 = lax.dot_general(g, wa_ref[...], (((1,), (0,)), ((), ())),
                          preferred_element_type=jnp.float32) + ba_row_ref[...]
    l16_ref[...] = l16

    @pl.when(i == 0)
    def _init_stats():
        den_s[...] = jnp.zeros_like(den_s)
        oc_s[...] = jnp.zeros_like(oc_s)

    seg = seg_ref[...]
    bag = lax.broadcasted_iota(jnp.int32, (N_BAGS, BLK), 0)
    oht = (bag == seg).astype(jnp.float32)
    e = jnp.exp(l16)
    den_s[...] += lax.dot_general(oht, e, (((1,), (0,)), ((), ())),
                                  preferred_element_type=jnp.float32)
    oc_s[...] += jnp.sum(oht, axis=1, keepdims=True)

    @pl.when(i == N_BLKS - 1)
    def _emit_stats():
        d = den_s[...]
        den_ref[...] = jnp.where(d == 0.0, 1.0, d)
        cnt_bc = jnp.broadcast_to(oc_s[...], (N_BAGS, N_BAGS))
        r = lax.broadcasted_iota(jnp.int32, (N_BAGS, N_BAGS), 0)
        c = lax.broadcasted_iota(jnp.int32, (N_BAGS, N_BAGS), 1)
        off_row = jnp.sum(jnp.where(r < c, cnt_bc, 0.0), axis=0, keepdims=True)
        cnt_row = jnp.sum(jnp.where(r == c, cnt_bc, 0.0), axis=0, keepdims=True)
        zeros = jnp.zeros((6, N_BAGS), jnp.float32)
        offcnt_ref[...] = jnp.concatenate(
            [off_row, cnt_row, zeros], axis=0).astype(jnp.int32)


def _compute_logits(x, seg_row, wvu, bvu, wa16, ba_row):
    return pl.pallas_call(
        _logits_body,
        grid=(N_BLKS,),
        in_specs=[
            pl.BlockSpec((BLK, EMBED), lambda i: (i, 0)),
            pl.BlockSpec((1, BLK), lambda i: (0, i)),
            pl.BlockSpec((2 * HIDDEN_PAD, EMBED), lambda i: (0, 0)),
            pl.BlockSpec((1, 2 * HIDDEN_PAD), lambda i: (0, 0)),
            pl.BlockSpec((HIDDEN_PAD, HP), lambda i: (0, 0)),
            pl.BlockSpec((1, HP), lambda i: (0, 0)),
        ],
        out_specs=[
            pl.BlockSpec((BLK, HP), lambda i: (i, 0)),
            pl.BlockSpec((BLK, EMBED), lambda i: (i, 0)),
            pl.BlockSpec((N_BAGS, HP), lambda i: (0, 0)),
            pl.BlockSpec((8, N_BAGS), lambda i: (0, 0)),
        ],
        out_shape=[
            jax.ShapeDtypeStruct((N_TOK, HP), jnp.float32),
            jax.ShapeDtypeStruct((N_TOK, EMBED), jnp.bfloat16),
            jax.ShapeDtypeStruct((N_BAGS, HP), jnp.float32),
            jax.ShapeDtypeStruct((8, N_BAGS), jnp.int32),
        ],
        scratch_shapes=[
            pltpu.VMEM((N_BAGS, HP), jnp.float32),
            pltpu.VMEM((N_BAGS, 1), jnp.float32),
        ],
    )(x, seg_row, wvu, bvu, wa16, ba_row)


def _pool_body(l16_ref, seg_ref, den_ref, x_ref, wm_ref, bm_ref, out_ref, acc_ref):
    i = pl.program_id(0)

    @pl.when(i == 0)
    def _init():
        acc_ref[...] = jnp.zeros_like(acc_ref)

    @pl.when(i < N_BLKS3)
    def _accum():
        l16 = l16_ref[...]
        seg = seg_ref[...]
        bag = lax.broadcasted_iota(jnp.int32, (N_BAGS, BLK3), 0)
        oht = (bag == seg).astype(jnp.float32)
        tok_den = lax.dot_general(oht, den_ref[...], (((0,), (0,)), ((), ())),
                                  preferred_element_type=jnp.float32)
        att16 = jnp.maximum(jnp.exp(l16) / tok_den, CLIP)
        p = lax.broadcasted_iota(jnp.int32, (HP, HEADS * N_BAGS), 0)
        q = lax.broadcasted_iota(jnp.int32, (HP, HEADS * N_BAGS), 1)
        expand = (p == q // N_BAGS).astype(jnp.float32)
        att64 = jnp.dot(att16, expand, preferred_element_type=jnp.float32)
        ps = lax.broadcasted_iota(jnp.int32, (N_BAGS, HEADS * N_BAGS), 0)
        qs = lax.broadcasted_iota(jnp.int32, (N_BAGS, HEADS * N_BAGS), 1)
        r2 = ((qs - (qs // N_BAGS) * N_BAGS) == ps).astype(jnp.float32)
        mask = lax.dot_general(oht, r2, (((0,), (0,)), ((), ())),
                               preferred_element_type=jnp.float32)
        a_mat = (att64 * mask).astype(jnp.bfloat16)
        acc_ref[...] += lax.dot_general(a_mat, x_ref[...], (((0,), (0,)), ((), ())),
                                        preferred_element_type=jnp.float32)

    @pl.when(i == N_BLKS3)
    def _final():
        acc = acc_ref[...]
        res = bm_ref[...]
        for h in range(HEADS):
            res += lax.dot_general(
                acc[h * N_BAGS:(h + 1) * N_BAGS, :],
                wm_ref[:, pl.ds(h * EMBED, EMBED)],
                (((1,), (1,)), ((), ())),
                preferred_element_type=jnp.float32)
        out_ref[...] = res


def _pool_project(l16, seg_row, den, xb, wm, bm2d):
    last = N_BLKS3 - 1
    return pl.pallas_call(
        _pool_body,
        grid=(N_BLKS3 + 1,),
        in_specs=[
            pl.BlockSpec((BLK3, HP), lambda i: (jnp.minimum(i, last), 0)),
            pl.BlockSpec((1, BLK3), lambda i: (0, jnp.minimum(i, last))),
            pl.BlockSpec((N_BAGS, HP), lambda i: (0, 0)),
            pl.BlockSpec((BLK3, EMBED), lambda i: (jnp.minimum(i, last), 0)),
            pl.BlockSpec((EMBED, HEADS * EMBED), lambda i: (0, 0)),
            pl.BlockSpec((N_BAGS, EMBED), lambda i: (0, 0)),
        ],
        out_specs=pl.BlockSpec((N_BAGS, EMBED), lambda i: (0, 0)),
        out_shape=jax.ShapeDtypeStruct((N_BAGS, EMBED), jnp.float32),
        scratch_shapes=[pltpu.VMEM((HEADS * N_BAGS, EMBED), jnp.float32)],
    )(l16, seg_row, den, xb, wm, bm2d)


def _w_body(l16_hbm, seg_hbm, off_hbm, cnt_hbm, den_hbm, w_hbm,
            seg_v, off_v, cnt_v, den_v, idx_v, col_v, d_v, rows_v, w_v, sem):
    c = lax.axis_index("c")
    s = lax.axis_index("s")
    wid = s * 2 + c
    base = wid * CHUNK
    pltpu.sync_copy(seg_hbm.at[pl.ds(base, CHUNK)], seg_v)
    pltpu.sync_copy(off_hbm, off_v)
    pltpu.sync_copy(cnt_hbm, cnt_v)
    pltpu.sync_copy(den_hbm, den_v)

    lane = lax.iota(jnp.int32, 16)

    def phase1(g, carry):
        i16 = g * 16 + lane
        q = i16 >> 2
        hh = i16 & 3
        sg = plsc.load_gather(seg_v, [q])
        off = plsc.load_gather(off_v, [sg])
        n = plsc.load_gather(cnt_v, [sg])
        k = (base + q - off) * HEADS + hh
        cdiv = k // n
        idx_v[pl.ds(g * 16, 16)] = off + (k - cdiv * n)
        col_v[pl.ds(g * 16, 16)] = cdiv
        d_v[pl.ds(g * 16, 16)] = plsc.load_gather(den_v, [sg, cdiv])
        return carry

    lax.fori_loop(0, ELEMS // 16, phase1, 0, unroll=4)

    def phase2(j, carry):
        pltpu.async_copy(l16_hbm.at[idx_v.at[pl.ds(j * DMA_B, DMA_B)]],
                         rows_v.at[pl.ds(j * DMA_B, DMA_B)], sem)
        return carry

    lax.fori_loop(0, ELEMS // DMA_B, phase2, 0, unroll=False)
    pltpu.make_async_copy(l16_hbm.at[pl.ds(0, ELEMS)], rows_v, sem).wait()

    def phase3(g, carry):
        i16 = g * 16 + lane
        cdiv = col_v[pl.ds(g * 16, 16)]
        lg = plsc.load_gather(rows_v, [i16, cdiv])
        d = d_v[pl.ds(g * 16, 16)]
        wv = jnp.maximum(jnp.exp(lg) / d, CLIP)
        plsc.store_scatter(w_v, [i16 & 3, i16 >> 2], wv)
        return carry

    lax.fori_loop(0, ELEMS // 16, phase3, 0, unroll=4)
    pltpu.sync_copy(w_v, w_hbm.at[:, pl.ds(base, CHUNK)])


@functools.lru_cache(maxsize=1)
def _get_w_kernel():
    @functools.partial(
        pl.kernel,
        mesh=plsc.VectorSubcoreMesh(core_axis_name="c", subcore_axis_name="s"),
        out_type=jax.ShapeDtypeStruct((HEADS, N_TOK), jnp.float32),
        compiler_params=pltpu.CompilerParams(
            needs_layout_passes=False, use_tc_tiling_on_sc=False),
        scratch_types=[
            pltpu.VMEM((CHUNK,), jnp.int32),
            pltpu.VMEM((N_BAGS,), jnp.int32),
            pltpu.VMEM((N_BAGS,), jnp.int32),
            pltpu.VMEM((N_BAGS, HP), jnp.float32),
            pltpu.VMEM((ELEMS,), jnp.int32),
            pltpu.VMEM((ELEMS,), jnp.int32),
            pltpu.VMEM((ELEMS,), jnp.float32),
            pltpu.VMEM((ELEMS, HP), jnp.float32),
            pltpu.VMEM((HEADS, CHUNK), jnp.float32),
            pltpu.SemaphoreType.DMA,
        ],
    )
    def _w_sc(l16, seg, off, cnt, den, w_out, *scratch):
        _w_body(l16, seg, off, cnt, den, w_out, *scratch)

    return _w_sc


def _w_sparsecore(l16, seg, off, cnt, den):
    return _get_w_kernel()(l16, seg, off, cnt, den)


def kernel(x, supercase_indices, Wv, bv, Wu, bu, Wa, ba, Wm, bm):
    seg = supercase_indices.astype(jnp.int32)
    seg_row = seg.reshape(1, N_TOK)

    h = Wv.shape[0]
    pad = HIDDEN_PAD - h
    zrow = jnp.zeros((pad, EMBED), jnp.float32)
    wvu = jnp.concatenate([Wv, zrow, Wu, zrow],
                          axis=0).astype(jnp.bfloat16)
    zb = jnp.zeros((pad,), jnp.float32)
    bvu = jnp.concatenate([bv, zb, bu, zb]).reshape(1, 2 * HIDDEN_PAD)
    wa16 = jnp.zeros((HIDDEN_PAD, HP), jnp.float32).at[:h, :HEADS].set(Wa.T)
    ba_row = jnp.zeros((1, HP), jnp.float32).at[0, :HEADS].set(ba)
    bm2d = jnp.broadcast_to(bm.reshape(1, EMBED), (N_BAGS, EMBED))

    l16, xb, den, offcnt = _compute_logits(x, seg_row, wvu, bvu, wa16, ba_row)
    out = _pool_project(l16, seg_row, den, xb, Wm, bm2d)
    wt = _w_sparsecore(l16, seg, offcnt[0], offcnt[1], den)
    return (out, wt.T)


# speedup vs baseline: 1.1248x; 1.0017x over previous
import functools

import jax
import jax.numpy as jnp
from jax import lax
from jax.experimental import pallas as pl
from jax.experimental.pallas import tpu as pltpu
from jax.experimental.pallas import tpu_sc as plsc

EMBED = 1024
HEADS = 4
HP = 16
N_TOK = 32768
N_BAGS = 16
HIDDEN_PAD = 384
BLK = 1024
N_BLKS = N_TOK // BLK
BLK3 = 4096
N_BLKS3 = N_TOK // BLK3
CLIP = 1e-5

NW = 32
CHUNK = N_TOK // NW
ELEMS = CHUNK * HEADS
DMA_B = 128


def _logits_body(x_ref, seg_ref, wvu_ref, bvu_ref, wa_ref, ba_row_ref,
                 l16_ref, xb_ref, den_ref, offcnt_ref, den_s, oc_s):
    i = pl.program_id(0)
    x = x_ref[...].astype(jnp.bfloat16)
    xb_ref[...] = x
    pre = lax.dot_general(x, wvu_ref[...], (((1,), (1,)), ((), ())),
                          preferred_element_type=jnp.float32)
    pre = pre + bvu_ref[...]
    v = jnp.tanh(pre[:, :HIDDEN_PAD])
    u = jax.nn.sigmoid(pre[:, HIDDEN_PAD:])
    g = v * u
    l16 = lax.dot_general(g, wa_ref[...], (((1,), (0,)), ((), ())),
                          preferred_element_type=jnp.float32) + ba_row_ref[...]
    l16_ref[...] = l16

    @pl.when(i == 0)
    def _init_stats():
        den_s[...] = jnp.zeros_like(den_s)
        oc_s[...] = jnp.zeros_like(oc_s)

    seg = seg_ref[...]
    bag = lax.broadcasted_iota(jnp.int32, (N_BAGS, BLK), 0)
    oht = (bag == seg).astype(jnp.float32)
    e = jnp.exp(l16)
    den_s[...] += lax.dot_general(oht, e, (((1,), (0,)), ((), ())),
                                  preferred_element_type=jnp.float32)
    oc_s[...] += jnp.sum(oht, axis=1, keepdims=True)

    @pl.when(i == N_BLKS - 1)
    def _emit_stats():
        d = den_s[...]
        den_ref[...] = jnp.where(d == 0.0, 1.0, d)
        cnt_bc = jnp.broadcast_to(oc_s[...], (N_BAGS, N_BAGS))
        r = lax.broadcasted_iota(jnp.int32, (N_BAGS, N_BAGS), 0)
        c = lax.broadcasted_iota(jnp.int32, (N_BAGS, N_BAGS), 1)
        off_row = jnp.sum(jnp.where(r < c, cnt_bc, 0.0), axis=0, keepdims=True)
        cnt_row = jnp.sum(jnp.where(r == c, cnt_bc, 0.0), axis=0, keepdims=True)
        zeros = jnp.zeros((6, N_BAGS), jnp.float32)
        offcnt_ref[...] = jnp.concatenate(
            [off_row, cnt_row, zeros], axis=0).astype(jnp.int32)


def _compute_logits(x, seg_row, wvu, bvu, wa16, ba_row):
    return pl.pallas_call(
        _logits_body,
        grid=(N_BLKS,),
        in_specs=[
            pl.BlockSpec((BLK, EMBED), lambda i: (i, 0)),
            pl.BlockSpec((1, BLK), lambda i: (0, i)),
            pl.BlockSpec((2 * HIDDEN_PAD, EMBED), lambda i: (0, 0)),
            pl.BlockSpec((1, 2 * HIDDEN_PAD), lambda i: (0, 0)),
            pl.BlockSpec((HIDDEN_PAD, HP), lambda i: (0, 0)),
            pl.BlockSpec((1, HP), lambda i: (0, 0)),
        ],
        out_specs=[
            pl.BlockSpec((BLK, HP), lambda i: (i, 0)),
            pl.BlockSpec((BLK, EMBED), lambda i: (i, 0)),
            pl.BlockSpec((N_BAGS, HP), lambda i: (0, 0)),
            pl.BlockSpec((8, N_BAGS), lambda i: (0, 0)),
        ],
        out_shape=[
            jax.ShapeDtypeStruct((N_TOK, HP), jnp.float32),
            jax.ShapeDtypeStruct((N_TOK, EMBED), jnp.bfloat16),
            jax.ShapeDtypeStruct((N_BAGS, HP), jnp.float32),
            jax.ShapeDtypeStruct((8, N_BAGS), jnp.int32),
        ],
        scratch_shapes=[
            pltpu.VMEM((N_BAGS, HP), jnp.float32),
            pltpu.VMEM((N_BAGS, 1), jnp.float32),
        ],
    )(x, seg_row, wvu, bvu, wa16, ba_row)


def _pool_body(l16_ref, seg_ref, den_ref, x_ref, wm_ref, bm_ref, out_ref, acc_ref):
    i = pl.program_id(0)

    @pl.when(i == 0)
    def _init():
        acc_ref[...] = jnp.zeros_like(acc_ref)

    @pl.when(i < N_BLKS3)
    def _accum():
        l16 = l16_ref[...]
        seg = seg_ref[...]
        bag = lax.broadcasted_iota(jnp.int32, (N_BAGS, BLK3), 0)
        oht = (bag == seg).astype(jnp.float32)
        tok_den = lax.dot_general(oht, den_ref[...], (((0,), (0,)), ((), ())),
                                  preferred_element_type=jnp.float32)
        att16 = jnp.maximum(jnp.exp(l16) / tok_den, CLIP)
        p = lax.broadcasted_iota(jnp.int32, (HP, HEADS * N_BAGS), 0)
        q = lax.broadcasted_iota(jnp.int32, (HP, HEADS * N_BAGS), 1)
        expand = (p == q // N_BAGS).astype(jnp.float32)
        att64 = jnp.dot(att16, expand, preferred_element_type=jnp.float32)
        ps = lax.broadcasted_iota(jnp.int32, (N_BAGS, HEADS * N_BAGS), 0)
        qs = lax.broadcasted_iota(jnp.int32, (N_BAGS, HEADS * N_BAGS), 1)
        r2 = ((qs - (qs // N_BAGS) * N_BAGS) == ps).astype(jnp.float32)
        mask = lax.dot_general(oht, r2, (((0,), (0,)), ((), ())),
                               preferred_element_type=jnp.float32)
        a_mat = (att64 * mask).astype(jnp.bfloat16)
        acc_ref[...] += lax.dot_general(a_mat, x_ref[...], (((0,), (0,)), ((), ())),
                                        preferred_element_type=jnp.float32)

    @pl.when(i == N_BLKS3)
    def _final():
        acc = acc_ref[...]
        res = bm_ref[...]
        for h in range(HEADS):
            res += lax.dot_general(
                acc[h * N_BAGS:(h + 1) * N_BAGS, :],
                wm_ref[:, pl.ds(h * EMBED, EMBED)],
                (((1,), (1,)), ((), ())),
                preferred_element_type=jnp.float32)
        out_ref[...] = res


def _pool_project(l16, seg_row, den, xb, wm, bm2d):
    last = N_BLKS3 - 1
    return pl.pallas_call(
        _pool_body,
        grid=(N_BLKS3 + 1,),
        in_specs=[
            pl.BlockSpec((BLK3, HP), lambda i: (jnp.minimum(i, last), 0)),
            pl.BlockSpec((1, BLK3), lambda i: (0, jnp.minimum(i, last))),
            pl.BlockSpec((N_BAGS, HP), lambda i: (0, 0)),
            pl.BlockSpec((BLK3, EMBED), lambda i: (jnp.minimum(i, last), 0)),
            pl.BlockSpec((EMBED, HEADS * EMBED), lambda i: (0, 0)),
            pl.BlockSpec((N_BAGS, EMBED), lambda i: (0, 0)),
        ],
        out_specs=pl.BlockSpec((N_BAGS, EMBED), lambda i: (0, 0)),
        out_shape=jax.ShapeDtypeStruct((N_BAGS, EMBED), jnp.float32),
        scratch_shapes=[pltpu.VMEM((HEADS * N_BAGS, EMBED), jnp.float32)],
    )(l16, seg_row, den, xb, wm, bm2d)


def _w_body(l16_hbm, seg_hbm, off_hbm, cnt_hbm, den_hbm, w_hbm,
            seg_v, off_v, cnt_v, den_v, idx_v, col_v, d_v, rows_v, w_v, sem):
    c = lax.axis_index("c")
    s = lax.axis_index("s")
    wid = s * 2 + c
    base = wid * CHUNK
    pltpu.sync_copy(seg_hbm.at[pl.ds(base, CHUNK)], seg_v)
    pltpu.sync_copy(off_hbm, off_v)
    pltpu.sync_copy(cnt_hbm, cnt_v)
    pltpu.sync_copy(den_hbm, den_v)

    lane = lax.iota(jnp.int32, 16)

    def phase1(g, carry):
        i16 = g * 16 + lane
        q = i16 >> 2
        hh = i16 & 3
        sg = plsc.load_gather(seg_v, [q])
        off = plsc.load_gather(off_v, [sg])
        n = plsc.load_gather(cnt_v, [sg])
        k = (base + q - off) * HEADS + hh
        kf = (k.astype(jnp.float32) + 0.5) / n.astype(jnp.float32)
        cdiv = kf.astype(jnp.int32)
        idx_v[pl.ds(g * 16, 16)] = off + (k - cdiv * n)
        col_v[pl.ds(g * 16, 16)] = cdiv
        d_v[pl.ds(g * 16, 16)] = plsc.load_gather(den_v, [sg, cdiv])
        return carry

    lax.fori_loop(0, ELEMS // 16, phase1, 0, unroll=4)

    def phase2(j, carry):
        pltpu.async_copy(l16_hbm.at[idx_v.at[pl.ds(j * DMA_B, DMA_B)]],
                         rows_v.at[pl.ds(j * DMA_B, DMA_B)], sem)
        return carry

    lax.fori_loop(0, ELEMS // DMA_B, phase2, 0, unroll=False)
    pltpu.make_async_copy(l16_hbm.at[pl.ds(0, ELEMS)], rows_v, sem).wait()

    def phase3(g, carry):
        i16 = g * 16 + lane
        cdiv = col_v[pl.ds(g * 16, 16)]
        lg = plsc.load_gather(rows_v, [i16, cdiv])
        d = d_v[pl.ds(g * 16, 16)]
        wv = jnp.maximum(jnp.exp(lg) / d, CLIP)
        plsc.store_scatter(w_v, [i16 & 3, i16 >> 2], wv)
        return carry

    lax.fori_loop(0, ELEMS // 16, phase3, 0, unroll=4)
    pltpu.sync_copy(w_v, w_hbm.at[:, pl.ds(base, CHUNK)])


@functools.lru_cache(maxsize=1)
def _get_w_kernel():
    @functools.partial(
        pl.kernel,
        mesh=plsc.VectorSubcoreMesh(core_axis_name="c", subcore_axis_name="s"),
        out_type=jax.ShapeDtypeStruct((HEADS, N_TOK), jnp.float32),
        compiler_params=pltpu.CompilerParams(
            needs_layout_passes=False, use_tc_tiling_on_sc=False),
        scratch_types=[
            pltpu.VMEM((CHUNK,), jnp.int32),
            pltpu.VMEM((N_BAGS,), jnp.int32),
            pltpu.VMEM((N_BAGS,), jnp.int32),
            pltpu.VMEM((N_BAGS, HP), jnp.float32),
            pltpu.VMEM((ELEMS,), jnp.int32),
            pltpu.VMEM((ELEMS,), jnp.int32),
            pltpu.VMEM((ELEMS,), jnp.float32),
            pltpu.VMEM((ELEMS, HP), jnp.float32),
            pltpu.VMEM((HEADS, CHUNK), jnp.float32),
            pltpu.SemaphoreType.DMA,
        ],
    )
    def _w_sc(l16, seg, off, cnt, den, w_out, *scratch):
        _w_body(l16, seg, off, cnt, den, w_out, *scratch)

    return _w_sc


def _w_sparsecore(l16, seg, off, cnt, den):
    return _get_w_kernel()(l16, seg, off, cnt, den)


def kernel(x, supercase_indices, Wv, bv, Wu, bu, Wa, ba, Wm, bm):
    seg = supercase_indices.astype(jnp.int32)
    seg_row = seg.reshape(1, N_TOK)

    h = Wv.shape[0]
    pad = HIDDEN_PAD - h
    zrow = jnp.zeros((pad, EMBED), jnp.float32)
    wvu = jnp.concatenate([Wv, zrow, Wu, zrow],
                          axis=0).astype(jnp.bfloat16)
    zb = jnp.zeros((pad,), jnp.float32)
    bvu = jnp.concatenate([bv, zb, bu, zb]).reshape(1, 2 * HIDDEN_PAD)
    wa16 = jnp.zeros((HIDDEN_PAD, HP), jnp.float32).at[:h, :HEADS].set(Wa.T)
    ba_row = jnp.zeros((1, HP), jnp.float32).at[0, :HEADS].set(ba)
    bm2d = jnp.broadcast_to(bm.reshape(1, EMBED), (N_BAGS, EMBED))

    l16, xb, den, offcnt = _compute_logits(x, seg_row, wvu, bvu, wa16, ba_row)
    out = _pool_project(l16, seg_row, den, xb, Wm, bm2d)
    wt = _w_sparsecore(l16, seg, offcnt[0], offcnt[1], den)
    return (out, wt.T)
